# Initial kernel scaffold; baseline (speedup 1.0000x reference)
import jax
import jax.numpy as jnp
from jax.experimental import pallas as pl


def kernel(inputs, embeddings):
    raise NotImplementedError("write your pallas kernel here")


# speedup vs baseline: 1.4927x; 1.4927x over previous
import functools

import jax
import jax.numpy as jnp
from jax import lax
from jax.experimental import pallas as pl
from jax.experimental.pallas import tpu as pltpu
from jax.experimental.pallas import tpu_sc as plsc

_VOCAB = 1000000
_HIDDEN = 32
_BATCH = 16384
_HIST = 20
_B = _BATCH * _HIST
_NC = 2
_NS = 16
_NW = _NC * _NS
_B_PER_W = _B // _NW
_CHUNK = 1024
_N_CHUNKS = _B_PER_W // _CHUNK


@functools.partial(
    pl.kernel,
    mesh=plsc.VectorSubcoreMesh(core_axis_name="c", subcore_axis_name="s"),
    out_type=jax.ShapeDtypeStruct((_B, _HIDDEN), jnp.float32),
    scratch_types=[
        pltpu.VMEM((_CHUNK,), jnp.int32),
        pltpu.VMEM((_CHUNK, _HIDDEN), jnp.float32),
        pltpu.SemaphoreType.DMA,
    ],
    compiler_params=pltpu.CompilerParams(use_tc_tiling_on_sc=False),
)
def _embed_gather(idx_hbm, table_hbm, out_hbm, idx_v, rows_v, sem):
    wid = lax.axis_index("s") * _NC + lax.axis_index("c")
    base = wid * _B_PER_W

    def body(i, carry):
        off = base + i * _CHUNK
        pltpu.sync_copy(idx_hbm.at[pl.ds(off, _CHUNK)], idx_v)
        pltpu.async_copy(table_hbm.at[idx_v], rows_v, sem).wait()
        pltpu.sync_copy(rows_v, out_hbm.at[pl.ds(off, _CHUNK)])
        return carry

    lax.fori_loop(0, _N_CHUNKS, body, 0)


def kernel(inputs, embeddings):
    flat_idx = inputs.reshape(-1).astype(jnp.int32)
    out = _embed_gather(flat_idx, embeddings)
    return out.reshape(_BATCH, _HIST, _HIDDEN)


# speedup vs baseline: 1.5137x; 1.0141x over previous
import functools

import jax
import jax.numpy as jnp
from jax import lax
from jax.experimental import pallas as pl
from jax.experimental.pallas import tpu as pltpu
from jax.experimental.pallas import tpu_sc as plsc

_VOCAB = 1000000
_HIDDEN = 32
_BATCH = 16384
_HIST = 20
_B = _BATCH * _HIST
_NC = 2
_NS = 16
_NW = _NC * _NS
_B_PER_W = _B // _NW
_CHUNK = 1024
_N_CHUNKS = _B_PER_W // _CHUNK
_NBUF = 3


@functools.partial(
    pl.kernel,
    mesh=plsc.VectorSubcoreMesh(core_axis_name="c", subcore_axis_name="s"),
    out_type=jax.ShapeDtypeStruct((_B, _HIDDEN), jnp.float32),
    scratch_types=[
        pltpu.VMEM((_N_CHUNKS, _CHUNK), jnp.int32),
        pltpu.VMEM((_NBUF, _CHUNK, _HIDDEN), jnp.float32),
        pltpu.SemaphoreType.DMA((_NBUF,)),
        pltpu.SemaphoreType.DMA((_NBUF,)),
    ],
    compiler_params=pltpu.CompilerParams(use_tc_tiling_on_sc=False),
)
def _embed_gather(idx_hbm, table_hbm, out_hbm, idx_v, rows_v, gsem, osem):
    wid = lax.axis_index("s") * _NC + lax.axis_index("c")
    base = wid * _B_PER_W

    pltpu.sync_copy(idx_hbm.at[pl.ds(wid * _N_CHUNKS, _N_CHUNKS)], idx_v)

    def start_gather(i):
        b = i % _NBUF
        return pltpu.async_copy(
            table_hbm.at[idx_v.at[i]], rows_v.at[b], gsem.at[b])

    def start_out(i):
        b = i % _NBUF
        return pltpu.async_copy(
            rows_v.at[b], out_hbm.at[pl.ds(base + i * _CHUNK, _CHUNK)],
            osem.at[b])

    gathers = [None] * _N_CHUNKS
    outs = [None] * _N_CHUNKS
    gathers[0] = start_gather(0)
    for i in range(_N_CHUNKS):
        nxt = i + 1
        if nxt < _N_CHUNKS:
            if nxt >= _NBUF:
                outs[nxt - _NBUF].wait()
            gathers[nxt] = start_gather(nxt)
        gathers[i].wait()
        outs[i] = start_out(i)
    for i in range(max(0, _N_CHUNKS - _NBUF), _N_CHUNKS):
        outs[i].wait()


def kernel(inputs, embeddings):
    flat_idx = inputs.reshape(_NW * _N_CHUNKS, _CHUNK).astype(jnp.int32)
    out = _embed_gather(flat_idx, embeddings)
    return out.reshape(_BATCH, _HIST, _HIDDEN)


# speedup vs baseline: 1.6091x; 1.0630x over previous
import functools

import jax
import jax.numpy as jnp
from jax import lax
from jax.experimental import pallas as pl
from jax.experimental.pallas import tpu as pltpu
from jax.experimental.pallas import tpu_sc as plsc

_VOCAB = 1000000
_HIDDEN = 32
_BATCH = 16384
_HIST = 20
_B = _BATCH * _HIST
_NC = 2
_NS = 16
_NW = _NC * _NS
_B_PER_W = _B // _NW
_CHUNK = 1024
_N_CHUNKS = _B_PER_W // _CHUNK
_NBUF = 3


@functools.partial(
    pl.kernel,
    mesh=plsc.VectorSubcoreMesh(core_axis_name="c", subcore_axis_name="s"),
    out_type=jax.ShapeDtypeStruct((_B, _HIDDEN), jnp.float32),
    scratch_types=[
        pltpu.VMEM((_N_CHUNKS, _CHUNK), jnp.int32),
        pltpu.VMEM((_NBUF, _CHUNK, _HIDDEN), jnp.float32),
        pltpu.SemaphoreType.DMA((_NBUF,)),
        pltpu.SemaphoreType.DMA((_NBUF,)),
    ],
    compiler_params=pltpu.CompilerParams(use_tc_tiling_on_sc=False),
)
def _embed_gather(idx_hbm, table_hbm, out_hbm, idx_v, rows_v, gsem, osem):
    wid = lax.axis_index("s") * _NC + lax.axis_index("c")
    base = wid * _B_PER_W

    pltpu.sync_copy(idx_hbm.at[pl.ds(wid * _N_CHUNKS, _N_CHUNKS)], idx_v)

    def start_gather(i):
        b = i % _NBUF
        return pltpu.async_copy(
            table_hbm.at[idx_v.at[i]], rows_v.at[b], gsem.at[b])

    def start_out(i):
        b = i % _NBUF
        return pltpu.async_copy(
            rows_v.at[b], out_hbm.at[pl.ds(base + i * _CHUNK, _CHUNK)],
            osem.at[b])

    gathers = [None] * _N_CHUNKS
    outs = [None] * _N_CHUNKS
    gathers[0] = start_gather(0)
    for i in range(_N_CHUNKS):
        nxt = i + 1
        if nxt < _N_CHUNKS:
            if nxt >= _NBUF:
                outs[nxt - _NBUF].wait()
            gathers[nxt] = start_gather(nxt)
        gathers[i].wait()
        outs[i] = start_out(i)
    for i in range(max(0, _N_CHUNKS - _NBUF), _N_CHUNKS):
        outs[i].wait()


_VB = 2048


_PACK = 128 // _HIDDEN


def _tc_transpose_body(emb_t_ref, out_ref):
    x = emb_t_ref[...]
    out_ref[...] = jnp.pad(x, ((0, 128 - _HIDDEN), (0, 0))).T


def _table_to_row_major(emb_t):
    grid = (_VOCAB + _VB - 1) // _VB
    return pl.pallas_call(
        _tc_transpose_body,
        grid=(grid,),
        in_specs=[pl.BlockSpec((_HIDDEN, _VB), lambda i: (0, i))],
        out_specs=pl.BlockSpec((_VB, 128), lambda i: (i, 0)),
        out_shape=jax.ShapeDtypeStruct((_VOCAB, 128), jnp.float32),
    )(emb_t)


def kernel(inputs, embeddings):
    flat_idx = (inputs.reshape(_NW * _N_CHUNKS, _CHUNK) * 4).astype(jnp.int32)
    emb_rm = _table_to_row_major(embeddings.T).reshape(_VOCAB * 4, _HIDDEN)
    out = _embed_gather(flat_idx, emb_rm)
    return out.reshape(_BATCH, _HIST, _HIDDEN)
